# baseline (speedup 1.0000x reference)
import functools
import math

import jax
import jax.numpy as jnp
from jax.experimental import pallas as pl
from jax.experimental.pallas import tpu as pltpu

_N = 4096
_BLK = 512
_NB = _N // _BLK
_RBLK = 1024
_NRB = _N // _RBLK
_HID = 256
_F32 = jnp.float32
_BF16 = jnp.bfloat16
_NEG = float("-inf")


def _pos_kernel(img_ref, wp_ref, bp_ref, out_ref):
    out_ref[...] = jnp.maximum(
        jnp.dot(img_ref[...], wp_ref[...], preferred_element_type=_F32) + bp_ref[...],
        0.0,
    )


def _sq_kernel(x_ref, out_ref):
    x = x_ref[...]
    out_ref[...] = jnp.sum(x * x, axis=1, keepdims=True)


def _h_kernel(x_ref, w_ref, out_ref):
    out_ref[...] = jnp.dot(x_ref[...], w_ref[...], preferred_element_type=_F32)


def _dist_kernel(sqc_ref, sqr_ref, xi_ref, xj_ref, d_ref, mx_ref):
    i = pl.program_id(0)
    j = pl.program_id(1)

    @pl.when(j <= i)
    def _():
        g = jax.lax.dot_general(
            xi_ref[...], xj_ref[...], (((1,), (1,)), ((), ())),
            preferred_element_type=_F32,
        )
        d = sqc_ref[...] + sqr_ref[...] - 2.0 * g
        d_ref[...] = d
        tile_max = jnp.max(d, axis=(0, 1), keepdims=True)
        first = (i == 0) & (j == 0)

        @pl.when(first)
        def _():
            mx_ref[...] = tile_max

        @pl.when(jnp.logical_not(first))
        def _():
            mx_ref[...] = jnp.maximum(mx_ref[...], tile_max)


def _adj_kernel(mx_ref, d_ref, m_ref, deg_ref):
    ib = pl.program_id(0)
    jb = pl.program_id(1)

    @pl.when(jb <= ib)
    def _():
        t = 0.5 * mx_ref[...]
        d = d_ref[...]
        ci = jax.lax.broadcasted_iota(jnp.int32, (_BLK, _BLK), 0) + ib * _BLK
        ri = jax.lax.broadcasted_iota(jnp.int32, (_BLK, _BLK), 1) + jb * _BLK
        a = jnp.where((d < t) & (ri < ci), 1.0, 0.0).astype(_F32)
        m_ref[...] = a.astype(_BF16)
        part = jnp.sum(a, axis=1, keepdims=True)

        @pl.when(jb == 0)
        def _():
            deg_ref[...] = part

        @pl.when(jb > 0)
        def _():
            deg_ref[...] += part

    @pl.when(jb > ib)
    def _():
        m_ref[...] = jnp.zeros_like(m_ref)


def _rmax(cb):
    return (cb * _BLK + _BLK - 1) // _RBLK


def _conv_kernel(mrow_ref, hr_ref, a_ref, mc_ref, hc_ref, b_ref, out_ref, deg_ref):
    cb = pl.program_id(0)
    rb = pl.program_id(1)

    @pl.when(rb == 0)
    def _():
        out_ref[...] = jnp.zeros_like(out_ref)

    @pl.when((cb == 0) & (rb == 0))
    def _():
        deg_ref[...] = jnp.zeros_like(deg_ref)

    @pl.when(rb <= _rmax(cb))
    def _():
        a = a_ref[...]
        part = jnp.dot(a, mrow_ref[...].astype(_BF16), preferred_element_type=_F32)
        deg_ref[pl.ds(cb * _BLK, _BLK), :] += part
        dinv_r = jax.lax.rsqrt(1.0 + deg_ref[pl.ds(rb * _RBLK, _RBLK), :])
        z = (dinv_r * hr_ref[...]).astype(_BF16)
        out_ref[...] += jnp.dot(a, z, preferred_element_type=_F32)

    @pl.when(rb == _NRB - 1)
    def _():
        dinv_c = jax.lax.rsqrt(1.0 + deg_ref[pl.ds(cb * _BLK, _BLK), :])
        h_c = hc_ref[...]
        y = dinv_c * out_ref[...] + (dinv_c * dinv_c) * h_c + b_ref[...]
        out_ref[...] = jnp.where(mc_ref[...] > 0, jnp.maximum(y, 0.0), 0.0)


def _conv1_kernel(degr_ref, hr_ref, a_ref, degc_ref, hc_ref, b_ref, out_ref):
    cb = pl.program_id(0)
    rb = pl.program_id(1)

    @pl.when(rb == 0)
    def _():
        out_ref[...] = jnp.zeros_like(out_ref)

    @pl.when(rb <= _rmax(cb))
    def _():
        dinv_r = jax.lax.rsqrt(1.0 + degr_ref[...])
        z = (dinv_r * hr_ref[...]).astype(_BF16)
        out_ref[...] += jnp.dot(a_ref[...], z, preferred_element_type=_F32)

    @pl.when(rb == _NRB - 1)
    def _():
        dinv_c = jax.lax.rsqrt(1.0 + degc_ref[...])
        h_c = hc_ref[...]
        y = dinv_c * out_ref[...] + (dinv_c * dinv_c) * h_c + b_ref[...]
        out_ref[...] = jnp.maximum(y, 0.0)


def _score_kernel(xr_ref, a_ref, xc_ref, wr_ref, br_ref, wt_ref, s_ref, acc_ref):
    cb = pl.program_id(0)
    rb = pl.program_id(1)

    @pl.when(rb == 0)
    def _():
        acc_ref[...] = jnp.zeros_like(acc_ref)

    @pl.when(rb <= _rmax(cb))
    def _():
        acc_ref[...] += jnp.dot(
            a_ref[...], xr_ref[...].astype(_BF16), preferred_element_type=_F32,
        )

    @pl.when(rb == _NRB - 1)
    def _():
        lin = (
            jnp.dot(acc_ref[...], wr_ref[...], preferred_element_type=_F32)
            + br_ref[...]
            + jnp.dot(xc_ref[...], wt_ref[...], preferred_element_type=_F32)
        )
        s_ref[...] = jnp.tanh(lin)


def _rank_pool_kernel(s_ref, m_ref, p_ref, sr_ref, mr_ref, pr_ref, x_ref, w_ref,
                      mnew_ref, pnew_ref, xn_ref, hn_ref, mx_ref, mn_ref, *, k):
    ib = pl.program_id(0)
    s_i = jnp.where(m_ref[...] > 0, s_ref[...], _NEG)
    s_j = jnp.where(mr_ref[...] > 0, sr_ref[...], _NEG)
    beats = (s_j > s_i) | ((s_j == s_i) & (pr_ref[...] < p_ref[...]))
    rank = jnp.sum(beats.astype(_F32), axis=1, keepdims=True)
    sel = (m_ref[...] > 0) & (rank < float(k))
    mnew_ref[...] = jnp.where(sel, 1.0, 0.0).astype(_F32)
    pnew_ref[...] = jnp.where(sel, rank, float(_N + 1)).astype(_F32)
    xn = jnp.where(sel, x_ref[...] * s_ref[...], 0.0)
    xn_ref[...] = xn
    hn_ref[...] = jnp.dot(xn, w_ref[...], preferred_element_type=_F32)
    tile_max = jnp.max(jnp.where(sel, xn, _NEG), axis=0, keepdims=True)
    tile_sum = jnp.sum(xn, axis=0, keepdims=True)

    @pl.when(ib == 0)
    def _():
        mx_ref[...] = tile_max
        mn_ref[...] = tile_sum

    @pl.when(ib > 0)
    def _():
        mx_ref[...] = jnp.maximum(mx_ref[...], tile_max)
        mn_ref[...] += tile_sum

    @pl.when(ib == _NB - 1)
    def _():
        mn_ref[...] = mn_ref[...] * (1.0 / float(k))


def _run_stage(M, h, deg1, m, posv, b, Wr, br, Wt, Wnext, k):
    rmin = lambda c, r: jnp.minimum(r, _rmax(c))
    if deg1 is not None:
        xc = pl.pallas_call(
            _conv1_kernel,
            grid=(_NB, _NRB),
            in_specs=[
                pl.BlockSpec((_RBLK, 1), lambda c, r: (rmin(c, r), 0)),
                pl.BlockSpec((_RBLK, _HID), lambda c, r: (rmin(c, r), 0)),
                pl.BlockSpec((_BLK, _RBLK), lambda c, r: (c, rmin(c, r))),
                pl.BlockSpec((_BLK, 1), lambda c, r: (c, 0)),
                pl.BlockSpec((_BLK, _HID), lambda c, r: (c, 0)),
                pl.BlockSpec((1, _HID), lambda c, r: (0, 0)),
            ],
            out_specs=pl.BlockSpec((_BLK, _HID), lambda c, r: (c, 0)),
            out_shape=jax.ShapeDtypeStruct((_N, _HID), _F32),
        )(deg1, h, M, deg1, h, b)
    else:
        xc = pl.pallas_call(
            _conv_kernel,
            grid=(_NB, _NRB),
            in_specs=[
                pl.BlockSpec((_RBLK, 1), lambda c, r: (rmin(c, r), 0)),
                pl.BlockSpec((_RBLK, _HID), lambda c, r: (rmin(c, r), 0)),
                pl.BlockSpec((_BLK, _RBLK), lambda c, r: (c, rmin(c, r))),
                pl.BlockSpec((_BLK, 1), lambda c, r: (c, 0)),
                pl.BlockSpec((_BLK, _HID), lambda c, r: (c, 0)),
                pl.BlockSpec((1, _HID), lambda c, r: (0, 0)),
            ],
            out_specs=pl.BlockSpec((_BLK, _HID), lambda c, r: (c, 0)),
            out_shape=jax.ShapeDtypeStruct((_N, _HID), _F32),
            scratch_shapes=[pltpu.VMEM((_N, 1), _F32)],
        )(m, h, M, m, h, b)

    s = pl.pallas_call(
        _score_kernel,
        grid=(_NB, _NRB),
        in_specs=[
            pl.BlockSpec((_RBLK, _HID), lambda c, r: (rmin(c, r), 0)),
            pl.BlockSpec((_BLK, _RBLK), lambda c, r: (c, rmin(c, r))),
            pl.BlockSpec((_BLK, _HID), lambda c, r: (c, 0)),
            pl.BlockSpec((_HID, 1), lambda c, r: (0, 0)),
            pl.BlockSpec((1, 1), lambda c, r: (0, 0)),
            pl.BlockSpec((_HID, 1), lambda c, r: (0, 0)),
        ],
        out_specs=pl.BlockSpec((_BLK, 1), lambda c, r: (c, 0)),
        out_shape=jax.ShapeDtypeStruct((_N, 1), _F32),
        scratch_shapes=[pltpu.VMEM((_BLK, _HID), _F32)],
    )(xc, M, xc, Wr, br, Wt)

    m2, p2, xn, hn, gmax, gmean = pl.pallas_call(
        functools.partial(_rank_pool_kernel, k=k),
        grid=(_NB,),
        in_specs=[
            pl.BlockSpec((_BLK, 1), lambda i: (i, 0)),
            pl.BlockSpec((_BLK, 1), lambda i: (i, 0)),
            pl.BlockSpec((_BLK, 1), lambda i: (i, 0)),
            pl.BlockSpec((1, _N), lambda i: (0, 0)),
            pl.BlockSpec((1, _N), lambda i: (0, 0)),
            pl.BlockSpec((1, _N), lambda i: (0, 0)),
            pl.BlockSpec((_BLK, _HID), lambda i: (i, 0)),
            pl.BlockSpec((_HID, _HID), lambda i: (0, 0)),
        ],
        out_specs=[
            pl.BlockSpec((_BLK, 1), lambda i: (i, 0)),
            pl.BlockSpec((_BLK, 1), lambda i: (i, 0)),
            pl.BlockSpec((_BLK, _HID), lambda i: (i, 0)),
            pl.BlockSpec((_BLK, _HID), lambda i: (i, 0)),
            pl.BlockSpec((1, _HID), lambda i: (0, 0)),
            pl.BlockSpec((1, _HID), lambda i: (0, 0)),
        ],
        out_shape=[
            jax.ShapeDtypeStruct((_N, 1), _F32),
            jax.ShapeDtypeStruct((_N, 1), _F32),
            jax.ShapeDtypeStruct((_N, _HID), _F32),
            jax.ShapeDtypeStruct((_N, _HID), _F32),
            jax.ShapeDtypeStruct((1, _HID), _F32),
            jax.ShapeDtypeStruct((1, _HID), _F32),
        ],
    )(s, m, posv, s.reshape(1, _N), m.reshape(1, _N), posv.reshape(1, _N), xc, Wnext)

    out = jnp.concatenate([gmax, gmean], axis=1)
    return xn, hn, m2, p2, out


def kernel(feature, img_info, W_pos, b_pos, W1, b1, W2, b2, W3, b3,
           Wr1, br1, Wt1, Wr2, br2, Wt2, Wr3, br3, Wt3):
    pos = pl.pallas_call(
        _pos_kernel,
        in_specs=[
            pl.BlockSpec((_N, 6), lambda: (0, 0)),
            pl.BlockSpec((6, 12), lambda: (0, 0)),
            pl.BlockSpec((1, 12), lambda: (0, 0)),
        ],
        out_specs=pl.BlockSpec((_N, 12), lambda: (0, 0)),
        out_shape=jax.ShapeDtypeStruct((_N, 12), _F32),
    )(img_info, W_pos, b_pos.reshape(1, 12))
    x0 = jnp.concatenate([feature, pos], axis=1)
    nf = x0.shape[1]

    sq = pl.pallas_call(
        _sq_kernel,
        grid=(_NB,),
        in_specs=[pl.BlockSpec((_BLK, nf), lambda i: (i, 0))],
        out_specs=pl.BlockSpec((_BLK, 1), lambda i: (i, 0)),
        out_shape=jax.ShapeDtypeStruct((_N, 1), _F32),
    )(x0)

    D, mx = pl.pallas_call(
        _dist_kernel,
        grid=(_NB, _NB),
        in_specs=[
            pl.BlockSpec((_BLK, 1), lambda i, j: (i, 0)),
            pl.BlockSpec((1, _BLK), lambda i, j: (0, jnp.minimum(j, i))),
            pl.BlockSpec((_BLK, nf), lambda i, j: (i, 0)),
            pl.BlockSpec((_BLK, nf), lambda i, j: (jnp.minimum(j, i), 0)),
        ],
        out_specs=[
            pl.BlockSpec((_BLK, _BLK), lambda i, j: (i, jnp.minimum(j, i))),
            pl.BlockSpec((1, 1), lambda i, j: (0, 0)),
        ],
        out_shape=[
            jax.ShapeDtypeStruct((_N, _N), _F32),
            jax.ShapeDtypeStruct((1, 1), _F32),
        ],
    )(sq, sq.reshape(1, _N), x0, x0)

    M, deg1 = pl.pallas_call(
        _adj_kernel,
        grid=(_NB, _NB),
        in_specs=[
            pl.BlockSpec((1, 1), lambda i, j: (0, 0)),
            pl.BlockSpec((_BLK, _BLK), lambda i, j: (i, jnp.minimum(j, i))),
        ],
        out_specs=[
            pl.BlockSpec((_BLK, _BLK), lambda i, j: (i, j)),
            pl.BlockSpec((_BLK, 1), lambda i, j: (i, 0)),
        ],
        out_shape=[
            jax.ShapeDtypeStruct((_N, _N), _BF16),
            jax.ShapeDtypeStruct((_N, 1), _F32),
        ],
    )(mx, D)

    h1 = pl.pallas_call(
        _h_kernel,
        grid=(_NB,),
        in_specs=[
            pl.BlockSpec((_BLK, nf), lambda i: (i, 0)),
            pl.BlockSpec((nf, _HID), lambda i: (0, 0)),
        ],
        out_specs=pl.BlockSpec((_BLK, _HID), lambda i: (i, 0)),
        out_shape=jax.ShapeDtypeStruct((_N, _HID), _F32),
    )(x0, W1)

    ones = jnp.ones((_N, 1), _F32)
    posv = jnp.arange(_N, dtype=_F32).reshape(_N, 1)

    k1 = int(math.ceil(0.75 * _N))
    k2 = int(math.ceil(0.75 * k1))
    k3 = int(math.ceil(0.75 * k2))

    x, h, m, posv, o1 = _run_stage(M, h1, deg1, ones, posv, b1.reshape(1, _HID),
                                   Wr1, br1.reshape(1, 1), Wt1, W2, k1)
    x, h, m, posv, o2 = _run_stage(M, h, None, m, posv, b2.reshape(1, _HID),
                                   Wr2, br2.reshape(1, 1), Wt2, W3, k2)
    x, h, m, posv, o3 = _run_stage(M, h, None, m, posv, b3.reshape(1, _HID),
                                   Wr3, br3.reshape(1, 1), Wt3, W3, k3)
    return o1 + o2 + o3


# speedup vs baseline: 1.0448x; 1.0448x over previous
import functools
import math

import jax
import jax.numpy as jnp
from jax.experimental import pallas as pl
from jax.experimental.pallas import tpu as pltpu

_N = 4096
_BLK = 512
_NB = _N // _BLK
_RBLK = 1024
_NRB = _N // _RBLK
_HID = 256
_F32 = jnp.float32
_BF16 = jnp.bfloat16
_NEG = float("-inf")


def _pos_kernel(img_ref, wp_ref, bp_ref, out_ref):
    out_ref[...] = jnp.maximum(
        jnp.dot(img_ref[...], wp_ref[...], preferred_element_type=_F32) + bp_ref[...],
        0.0,
    )


def _sq_kernel(x_ref, out_ref):
    x = x_ref[...]
    out_ref[...] = jnp.sum(x * x, axis=1, keepdims=True)


def _h_kernel(x_ref, w_ref, out_ref):
    out_ref[...] = jnp.dot(x_ref[...], w_ref[...], preferred_element_type=_F32)


def _dist_kernel(sqc_ref, sqr_ref, xi_ref, xj_ref, d_ref, mx_ref):
    i = pl.program_id(0)
    j = pl.program_id(1)

    @pl.when(j <= i)
    def _():
        g = jax.lax.dot_general(
            xi_ref[...], xj_ref[...], (((1,), (1,)), ((), ())),
            preferred_element_type=_F32,
        )
        d = sqc_ref[...] + sqr_ref[...] - 2.0 * g
        d_ref[...] = d
        tile_max = jnp.max(d, axis=(0, 1), keepdims=True)
        first = (i == 0) & (j == 0)

        @pl.when(first)
        def _():
            mx_ref[...] = tile_max

        @pl.when(jnp.logical_not(first))
        def _():
            mx_ref[...] = jnp.maximum(mx_ref[...], tile_max)


def _adj_kernel(mx_ref, d_ref, m_ref, deg_ref):
    ib = pl.program_id(0)
    jb = pl.program_id(1)

    @pl.when(jb <= ib)
    def _():
        t = 0.5 * mx_ref[...]
        d = d_ref[...]
        ci = jax.lax.broadcasted_iota(jnp.int32, (_BLK, _BLK), 0) + ib * _BLK
        ri = jax.lax.broadcasted_iota(jnp.int32, (_BLK, _BLK), 1) + jb * _BLK
        a = jnp.where((d < t) & (ri < ci), 1.0, 0.0).astype(_F32)
        m_ref[...] = a.astype(_BF16)
        part = jnp.sum(a, axis=1, keepdims=True)

        @pl.when(jb == 0)
        def _():
            deg_ref[...] = part

        @pl.when(jb > 0)
        def _():
            deg_ref[...] += part

    @pl.when(jb > ib)
    def _():
        m_ref[...] = jnp.zeros_like(m_ref)


def _rmax(cb):
    return (cb * _BLK + _BLK - 1) // _RBLK


def _conv_kernel(mrow_ref, hr_ref, a_ref, mc_ref, hc_ref, b_ref, out_ref, deg_ref):
    cb = pl.program_id(0)
    rb = pl.program_id(1)

    @pl.when(rb == 0)
    def _():
        out_ref[...] = jnp.zeros_like(out_ref)

    @pl.when((cb == 0) & (rb == 0))
    def _():
        deg_ref[...] = jnp.zeros_like(deg_ref)

    @pl.when(rb <= _rmax(cb))
    def _():
        a = a_ref[...]
        part = jnp.sum(a.astype(_F32) * mrow_ref[...], axis=1, keepdims=True)
        deg_ref[pl.ds(cb * _BLK, _BLK), :] += part
        dinv_r = jax.lax.rsqrt(1.0 + deg_ref[pl.ds(rb * _RBLK, _RBLK), :])
        z = (dinv_r * hr_ref[...]).astype(_BF16)
        out_ref[...] += jnp.dot(a, z, preferred_element_type=_F32)

    @pl.when(rb == _NRB - 1)
    def _():
        dinv_c = jax.lax.rsqrt(1.0 + deg_ref[pl.ds(cb * _BLK, _BLK), :])
        h_c = hc_ref[...]
        y = dinv_c * out_ref[...] + (dinv_c * dinv_c) * h_c + b_ref[...]
        out_ref[...] = jnp.where(mc_ref[...] > 0, jnp.maximum(y, 0.0), 0.0)


def _conv1_kernel(degr_ref, hr_ref, a_ref, degc_ref, hc_ref, b_ref, out_ref):
    cb = pl.program_id(0)
    rb = pl.program_id(1)

    @pl.when(rb == 0)
    def _():
        out_ref[...] = jnp.zeros_like(out_ref)

    @pl.when(rb <= _rmax(cb))
    def _():
        dinv_r = jax.lax.rsqrt(1.0 + degr_ref[...])
        z = (dinv_r * hr_ref[...]).astype(_BF16)
        out_ref[...] += jnp.dot(a_ref[...], z, preferred_element_type=_F32)

    @pl.when(rb == _NRB - 1)
    def _():
        dinv_c = jax.lax.rsqrt(1.0 + degc_ref[...])
        h_c = hc_ref[...]
        y = dinv_c * out_ref[...] + (dinv_c * dinv_c) * h_c + b_ref[...]
        out_ref[...] = jnp.maximum(y, 0.0)


def _score_kernel(xr_ref, a_ref, xc_ref, wr_ref, br_ref, wt_ref, s_ref, acc_ref):
    cb = pl.program_id(0)
    rb = pl.program_id(1)

    @pl.when(rb == 0)
    def _():
        acc_ref[...] = jnp.zeros_like(acc_ref)

    @pl.when(rb <= _rmax(cb))
    def _():
        acc_ref[...] += jnp.dot(
            a_ref[...], xr_ref[...].astype(_BF16), preferred_element_type=_F32,
        )

    @pl.when(rb == _NRB - 1)
    def _():
        lin = (
            jnp.dot(acc_ref[...], wr_ref[...], preferred_element_type=_F32)
            + br_ref[...]
            + jnp.dot(xc_ref[...], wt_ref[...], preferred_element_type=_F32)
        )
        s_ref[...] = jnp.tanh(lin)


def _rank_pool_kernel(s_ref, m_ref, p_ref, sr_ref, mr_ref, pr_ref, x_ref, w_ref,
                      mnew_ref, pnew_ref, xn_ref, hn_ref, mx_ref, mn_ref, *, k):
    ib = pl.program_id(0)
    s_i = jnp.where(m_ref[...] > 0, s_ref[...], _NEG)
    s_j = jnp.where(mr_ref[...] > 0, sr_ref[...], _NEG)
    beats = (s_j > s_i) | ((s_j == s_i) & (pr_ref[...] < p_ref[...]))
    rank = jnp.sum(beats.astype(_F32), axis=1, keepdims=True)
    sel = (m_ref[...] > 0) & (rank < float(k))
    mnew_ref[...] = jnp.where(sel, 1.0, 0.0).astype(_F32)
    pnew_ref[...] = jnp.where(sel, rank, float(_N + 1)).astype(_F32)
    xn = jnp.where(sel, x_ref[...] * s_ref[...], 0.0)
    xn_ref[...] = xn
    hn_ref[...] = jnp.dot(xn, w_ref[...], preferred_element_type=_F32)
    tile_max = jnp.max(jnp.where(sel, xn, _NEG), axis=0, keepdims=True)
    tile_sum = jnp.sum(xn, axis=0, keepdims=True)

    @pl.when(ib == 0)
    def _():
        mx_ref[...] = tile_max
        mn_ref[...] = tile_sum

    @pl.when(ib > 0)
    def _():
        mx_ref[...] = jnp.maximum(mx_ref[...], tile_max)
        mn_ref[...] += tile_sum

    @pl.when(ib == _NB - 1)
    def _():
        mn_ref[...] = mn_ref[...] * (1.0 / float(k))


def _run_stage(M, h, deg1, m, posv, b, Wr, br, Wt, Wnext, k):
    rmin = lambda c, r: jnp.minimum(r, _rmax(c))
    if deg1 is not None:
        xc = pl.pallas_call(
            _conv1_kernel,
            grid=(_NB, _NRB),
            in_specs=[
                pl.BlockSpec((_RBLK, 1), lambda c, r: (rmin(c, r), 0)),
                pl.BlockSpec((_RBLK, _HID), lambda c, r: (rmin(c, r), 0)),
                pl.BlockSpec((_BLK, _RBLK), lambda c, r: (c, rmin(c, r))),
                pl.BlockSpec((_BLK, 1), lambda c, r: (c, 0)),
                pl.BlockSpec((_BLK, _HID), lambda c, r: (c, 0)),
                pl.BlockSpec((1, _HID), lambda c, r: (0, 0)),
            ],
            out_specs=pl.BlockSpec((_BLK, _HID), lambda c, r: (c, 0)),
            out_shape=jax.ShapeDtypeStruct((_N, _HID), _F32),
        )(deg1, h, M, deg1, h, b)
    else:
        xc = pl.pallas_call(
            _conv_kernel,
            grid=(_NB, _NRB),
            in_specs=[
                pl.BlockSpec((1, _RBLK), lambda c, r: (0, rmin(c, r))),
                pl.BlockSpec((_RBLK, _HID), lambda c, r: (rmin(c, r), 0)),
                pl.BlockSpec((_BLK, _RBLK), lambda c, r: (c, rmin(c, r))),
                pl.BlockSpec((_BLK, 1), lambda c, r: (c, 0)),
                pl.BlockSpec((_BLK, _HID), lambda c, r: (c, 0)),
                pl.BlockSpec((1, _HID), lambda c, r: (0, 0)),
            ],
            out_specs=pl.BlockSpec((_BLK, _HID), lambda c, r: (c, 0)),
            out_shape=jax.ShapeDtypeStruct((_N, _HID), _F32),
            scratch_shapes=[pltpu.VMEM((_N, 1), _F32)],
        )(m.reshape(1, _N), h, M, m, h, b)

    s = pl.pallas_call(
        _score_kernel,
        grid=(_NB, _NRB),
        in_specs=[
            pl.BlockSpec((_RBLK, _HID), lambda c, r: (rmin(c, r), 0)),
            pl.BlockSpec((_BLK, _RBLK), lambda c, r: (c, rmin(c, r))),
            pl.BlockSpec((_BLK, _HID), lambda c, r: (c, 0)),
            pl.BlockSpec((_HID, 1), lambda c, r: (0, 0)),
            pl.BlockSpec((1, 1), lambda c, r: (0, 0)),
            pl.BlockSpec((_HID, 1), lambda c, r: (0, 0)),
        ],
        out_specs=pl.BlockSpec((_BLK, 1), lambda c, r: (c, 0)),
        out_shape=jax.ShapeDtypeStruct((_N, 1), _F32),
        scratch_shapes=[pltpu.VMEM((_BLK, _HID), _F32)],
    )(xc, M, xc, Wr, br, Wt)

    m2, p2, xn, hn, gmax, gmean = pl.pallas_call(
        functools.partial(_rank_pool_kernel, k=k),
        grid=(_NB,),
        in_specs=[
            pl.BlockSpec((_BLK, 1), lambda i: (i, 0)),
            pl.BlockSpec((_BLK, 1), lambda i: (i, 0)),
            pl.BlockSpec((_BLK, 1), lambda i: (i, 0)),
            pl.BlockSpec((1, _N), lambda i: (0, 0)),
            pl.BlockSpec((1, _N), lambda i: (0, 0)),
            pl.BlockSpec((1, _N), lambda i: (0, 0)),
            pl.BlockSpec((_BLK, _HID), lambda i: (i, 0)),
            pl.BlockSpec((_HID, _HID), lambda i: (0, 0)),
        ],
        out_specs=[
            pl.BlockSpec((_BLK, 1), lambda i: (i, 0)),
            pl.BlockSpec((_BLK, 1), lambda i: (i, 0)),
            pl.BlockSpec((_BLK, _HID), lambda i: (i, 0)),
            pl.BlockSpec((_BLK, _HID), lambda i: (i, 0)),
            pl.BlockSpec((1, _HID), lambda i: (0, 0)),
            pl.BlockSpec((1, _HID), lambda i: (0, 0)),
        ],
        out_shape=[
            jax.ShapeDtypeStruct((_N, 1), _F32),
            jax.ShapeDtypeStruct((_N, 1), _F32),
            jax.ShapeDtypeStruct((_N, _HID), _F32),
            jax.ShapeDtypeStruct((_N, _HID), _F32),
            jax.ShapeDtypeStruct((1, _HID), _F32),
            jax.ShapeDtypeStruct((1, _HID), _F32),
        ],
    )(s, m, posv, s.reshape(1, _N), m.reshape(1, _N), posv.reshape(1, _N), xc, Wnext)

    out = jnp.concatenate([gmax, gmean], axis=1)
    return xn, hn, m2, p2, out


def kernel(feature, img_info, W_pos, b_pos, W1, b1, W2, b2, W3, b3,
           Wr1, br1, Wt1, Wr2, br2, Wt2, Wr3, br3, Wt3):
    pos = pl.pallas_call(
        _pos_kernel,
        in_specs=[
            pl.BlockSpec((_N, 6), lambda: (0, 0)),
            pl.BlockSpec((6, 12), lambda: (0, 0)),
            pl.BlockSpec((1, 12), lambda: (0, 0)),
        ],
        out_specs=pl.BlockSpec((_N, 12), lambda: (0, 0)),
        out_shape=jax.ShapeDtypeStruct((_N, 12), _F32),
    )(img_info, W_pos, b_pos.reshape(1, 12))
    x0 = jnp.concatenate([feature, pos], axis=1)
    nf = x0.shape[1]

    sq = pl.pallas_call(
        _sq_kernel,
        grid=(_NB,),
        in_specs=[pl.BlockSpec((_BLK, nf), lambda i: (i, 0))],
        out_specs=pl.BlockSpec((_BLK, 1), lambda i: (i, 0)),
        out_shape=jax.ShapeDtypeStruct((_N, 1), _F32),
    )(x0)

    D, mx = pl.pallas_call(
        _dist_kernel,
        grid=(_NB, _NB),
        in_specs=[
            pl.BlockSpec((_BLK, 1), lambda i, j: (i, 0)),
            pl.BlockSpec((1, _BLK), lambda i, j: (0, jnp.minimum(j, i))),
            pl.BlockSpec((_BLK, nf), lambda i, j: (i, 0)),
            pl.BlockSpec((_BLK, nf), lambda i, j: (jnp.minimum(j, i), 0)),
        ],
        out_specs=[
            pl.BlockSpec((_BLK, _BLK), lambda i, j: (i, jnp.minimum(j, i))),
            pl.BlockSpec((1, 1), lambda i, j: (0, 0)),
        ],
        out_shape=[
            jax.ShapeDtypeStruct((_N, _N), _F32),
            jax.ShapeDtypeStruct((1, 1), _F32),
        ],
    )(sq, sq.reshape(1, _N), x0, x0)

    M, deg1 = pl.pallas_call(
        _adj_kernel,
        grid=(_NB, _NB),
        in_specs=[
            pl.BlockSpec((1, 1), lambda i, j: (0, 0)),
            pl.BlockSpec((_BLK, _BLK), lambda i, j: (i, jnp.minimum(j, i))),
        ],
        out_specs=[
            pl.BlockSpec((_BLK, _BLK), lambda i, j: (i, j)),
            pl.BlockSpec((_BLK, 1), lambda i, j: (i, 0)),
        ],
        out_shape=[
            jax.ShapeDtypeStruct((_N, _N), _BF16),
            jax.ShapeDtypeStruct((_N, 1), _F32),
        ],
    )(mx, D)

    h1 = pl.pallas_call(
        _h_kernel,
        grid=(_NB,),
        in_specs=[
            pl.BlockSpec((_BLK, nf), lambda i: (i, 0)),
            pl.BlockSpec((nf, _HID), lambda i: (0, 0)),
        ],
        out_specs=pl.BlockSpec((_BLK, _HID), lambda i: (i, 0)),
        out_shape=jax.ShapeDtypeStruct((_N, _HID), _F32),
    )(x0, W1)

    ones = jnp.ones((_N, 1), _F32)
    posv = jnp.arange(_N, dtype=_F32).reshape(_N, 1)

    k1 = int(math.ceil(0.75 * _N))
    k2 = int(math.ceil(0.75 * k1))
    k3 = int(math.ceil(0.75 * k2))

    x, h, m, posv, o1 = _run_stage(M, h1, deg1, ones, posv, b1.reshape(1, _HID),
                                   Wr1, br1.reshape(1, 1), Wt1, W2, k1)
    x, h, m, posv, o2 = _run_stage(M, h, None, m, posv, b2.reshape(1, _HID),
                                   Wr2, br2.reshape(1, 1), Wt2, W3, k2)
    x, h, m, posv, o3 = _run_stage(M, h, None, m, posv, b3.reshape(1, _HID),
                                   Wr3, br3.reshape(1, 1), Wt3, W3, k3)
    return o1 + o2 + o3
